# baseline (speedup 1.0000x reference)
import functools

import jax
import jax.numpy as jnp
from jax import lax
from jax.experimental import pallas as pl
from jax.experimental.pallas import tpu as pltpu
from jax.experimental.pallas import tpu_sc as plsc

NC = 2
NS = 16
LANES = 16
CHUNK = 128


def _sc_mesh():
  return plsc.VectorSubcoreMesh(
      core_axis_name="c", subcore_axis_name="s", num_cores=NC, num_subcores=NS
  )


def _make_deg_kernel(n, e):
  nw = NC * NS
  per_w = e // nw
  full = per_w // LANES
  rem = per_w - full * LANES
  buf = (full + (1 if rem else 0)) * LANES

  @functools.partial(
      pl.kernel,
      out_type=jax.ShapeDtypeStruct((nw, n), jnp.float32),
      mesh=_sc_mesh(),
      compiler_params=pltpu.CompilerParams(needs_layout_passes=False, use_tc_tiling_on_sc=False),
      scratch_types=[
          pltpu.VMEM((buf,), jnp.int32),
          pltpu.VMEM((n,), jnp.float32),
      ],
  )
  def deg_kernel(dst_hbm, hist_out, dst_v, hist_v):
    cid = lax.axis_index("c")
    sid = lax.axis_index("s")
    w = sid * NC + cid

    def zero_body(i, carry):
      hist_v[pl.ds(i * LANES, LANES)] = jnp.zeros((LANES,), jnp.float32)
      return carry

    lax.fori_loop(0, n // LANES, zero_body, 0)
    pltpu.sync_copy(dst_hbm.at[pl.ds(w * per_w, per_w)],
                    dst_v.at[pl.ds(0, per_w)])
    ones = jnp.ones((LANES,), jnp.float32)

    def body(i, carry):
      idx = dst_v[pl.ds(i * LANES, LANES)]
      plsc.addupdate_scatter(hist_v, [idx], ones)
      return carry

    lax.fori_loop(0, full, body, 0)
    if rem:
      idx = dst_v[pl.ds(full * LANES, LANES)]
      idx = jnp.clip(idx, 0, n - 1)
      m = lax.iota(jnp.int32, LANES) < rem
      plsc.addupdate_scatter(hist_v, [idx], ones, mask=m)
    pltpu.sync_copy(hist_v, hist_out.at[w])

  return deg_kernel


def _mm_body(x_ref, w_ref, hist_ref, ones_ref, y_ref, dinv_ref):
  deg = lax.dot_general(
      hist_ref[...], ones_ref[...],
      dimension_numbers=(((0,), (0,)), ((), ())),
      preferred_element_type=jnp.float32,
  )
  dinv = lax.rsqrt(deg + 1.0)
  y = jnp.dot(x_ref[...], w_ref[...], preferred_element_type=jnp.float32)
  y_ref[...] = y * dinv
  dinv_ref[...] = dinv


def _make_mm_call(n, d_in):
  nw = NC * NS
  nchunks = 4
  return pl.pallas_call(
      _mm_body,
      grid=(nchunks,),
      in_specs=[
          pl.BlockSpec((n, d_in), lambda c: (0, 0)),
          pl.BlockSpec((d_in, CHUNK), lambda c: (0, c)),
          pl.BlockSpec((nw, n), lambda c: (0, 0)),
          pl.BlockSpec((nw, 1), lambda c: (0, 0)),
      ],
      out_specs=[
          pl.BlockSpec((n, CHUNK), lambda c: (c, 0)),
          pl.BlockSpec((n, 1), lambda c: (0, 0)),
      ],
      out_shape=[
          jax.ShapeDtypeStruct((nchunks * n, CHUNK), jnp.float32),
          jax.ShapeDtypeStruct((n, 1), jnp.float32),
      ],
  )


def _make_scatter_kernel(n, e):
  eb = 80
  rows = e // eb
  tpr = rows // NS
  own = n // NS
  assert rows * eb == e and tpr * NS == rows
  assert (tpr - 2) % 3 == 0

  @functools.partial(
      pl.kernel,
      out_type=jax.ShapeDtypeStruct((4 * n, CHUNK), jnp.float32),
      mesh=_sc_mesh(),
      compiler_params=pltpu.CompilerParams(needs_layout_passes=False, use_tc_tiling_on_sc=False),
      scratch_types=[
          pltpu.VMEM((tpr * eb,), jnp.int32),
          pltpu.VMEM((tpr, eb), jnp.int32),
          pltpu.VMEM((eb, CHUNK), jnp.float32),
          pltpu.VMEM((eb, CHUNK), jnp.float32),
          pltpu.VMEM((eb, CHUNK), jnp.float32),
          pltpu.VMEM_SHARED((n, CHUNK), jnp.float32),
          pltpu.SemaphoreType.DMA,
          pltpu.SemaphoreType.DMA,
          pltpu.SemaphoreType.DMA,
          pltpu.SemaphoreType.DMA,
          pltpu.SemaphoreType.DMA,
          pltpu.SemaphoreType.DMA,
      ],
  )
  def scatter_kernel(y_hbm, src_hbm, dst_hbm, z_out,
                     src_v, dst_v, gb0, gb1, gb2, z_sh,
                     g0, g1, g2, s0, s1, s2):
    cid = lax.axis_index("c")
    sid = lax.axis_index("s")
    row_lo = sid * tpr
    gbufs = (gb0, gb1, gb2)
    gsems = (g0, g1, g2)
    ssems = (s0, s1, s2)
    pltpu.sync_copy(src_hbm.at[pl.ds(row_lo * eb, tpr * eb)], src_v)
    pltpu.sync_copy(dst_hbm.at[pl.ds(row_lo, tpr)], dst_v)

    def gwait(r):
      pltpu.make_async_copy(y_hbm.at[pl.ds(0, eb)], gbufs[r], gsems[r]).wait()

    def swait(r):
      pltpu.make_async_copy(y_hbm.at[pl.ds(0, eb)], gbufs[r], ssems[r]).wait()

    for p in range(2):
      c = 2 * cid + p
      base = c * n
      table = y_hbm.at[pl.ds(base, n)]

      def gather(j, r):
        pltpu.async_copy(table.at[src_v.at[pl.ds(j * eb, eb)]],
                         gbufs[r], gsems[r])

      def scatter(j, r):
        pltpu.async_copy(gbufs[r], z_sh.at[dst_v.at[j]], ssems[r], add=True)

      pltpu.sync_copy(y_hbm.at[pl.ds(base + sid * own, own)],
                      z_sh.at[pl.ds(sid * own, own)])
      plsc.subcore_barrier()

      gather(0, 0)
      gather(1, 1)
      gather(2, 2)
      gwait(0)
      scatter(0, 0)
      swait(0)
      gather(3, 0)
      gwait(1)
      scatter(1, 1)

      def body(k, carry):
        jb = 2 + 3 * k
        for m in range(3):
          j = jb + m
          r = (2 + m) % 3
          rn = (r + 2) % 3
          swait(rn)

          @pl.when(j + 2 < tpr)
          def _():
            gather(j + 2, rn)

          gwait(r)
          scatter(j, r)
        return carry

      lax.fori_loop(0, (tpr - 2) // 3, body, 0)
      swait((tpr - 1) % 3)
      plsc.subcore_barrier()
      pltpu.sync_copy(z_sh.at[pl.ds(sid * own, own)],
                      z_out.at[pl.ds(base + sid * own, own)])

  return scatter_kernel


def _final_body(z_ref, dinv_ref, bmu_ref, bls_ref, mu_ref, ls_ref):
  z = z_ref[...]
  dinv = dinv_ref[...]
  mu_ref[...] = jnp.concatenate([z[0], z[1]], axis=-1) * dinv + bmu_ref[...]
  ls_ref[...] = jnp.concatenate([z[2], z[3]], axis=-1) * dinv + bls_ref[...]


def _make_final_call(n, d_out):
  bm = 1000
  return pl.pallas_call(
      _final_body,
      grid=(n // bm,),
      in_specs=[
          pl.BlockSpec((4, bm, CHUNK), lambda i: (0, i, 0)),
          pl.BlockSpec((bm, 1), lambda i: (i, 0)),
          pl.BlockSpec((1, d_out), lambda i: (0, 0)),
          pl.BlockSpec((1, d_out), lambda i: (0, 0)),
      ],
      out_specs=[
          pl.BlockSpec((bm, d_out), lambda i: (i, 0)),
          pl.BlockSpec((bm, d_out), lambda i: (i, 0)),
      ],
      out_shape=[
          jax.ShapeDtypeStruct((n, d_out), jnp.float32),
          jax.ShapeDtypeStruct((n, d_out), jnp.float32),
      ],
  )


def kernel(x, edge_index, W_mu, b_mu, W_logstd, b_logstd):
  n, d_in = x.shape
  e = edge_index.shape[1]
  d_out = W_mu.shape[1]
  nw = NC * NS

  src = edge_index[0]
  dst = edge_index[1]
  w_cat = jnp.concatenate([W_mu, W_logstd], axis=1)

  hist = _make_deg_kernel(n, e)(dst)
  ones = jnp.ones((nw, 1), jnp.float32)
  y_flat, dinv = _make_mm_call(n, d_in)(x, w_cat, hist, ones)
  dst2 = dst.reshape(e // 80, 80)
  z_flat = _make_scatter_kernel(n, e)(y_flat, src, dst2)
  z4 = z_flat.reshape(4, n, CHUNK)
  bmu2 = b_mu.reshape(1, d_out)
  bls2 = b_logstd.reshape(1, d_out)
  mu, logstd = _make_final_call(n, d_out)(z4, dinv, bmu2, bls2)
  return (mu, logstd)


# speedup vs baseline: 1.0829x; 1.0829x over previous
import functools

import jax
import jax.numpy as jnp
from jax import lax
from jax.experimental import pallas as pl
from jax.experimental.pallas import tpu as pltpu
from jax.experimental.pallas import tpu_sc as plsc

NC = 2
NS = 16
LANES = 16
CHUNK = 128


def _sc_mesh():
  return plsc.VectorSubcoreMesh(
      core_axis_name="c", subcore_axis_name="s", num_cores=NC, num_subcores=NS
  )


def _make_deg_kernel(n, e):
  nw = NC * NS
  per_w = e // nw
  full = per_w // LANES
  rem = per_w - full * LANES
  buf = (full + (1 if rem else 0)) * LANES

  @functools.partial(
      pl.kernel,
      out_type=jax.ShapeDtypeStruct((nw, n), jnp.float32),
      mesh=_sc_mesh(),
      compiler_params=pltpu.CompilerParams(needs_layout_passes=False, use_tc_tiling_on_sc=False),
      scratch_types=[
          pltpu.VMEM((buf,), jnp.int32),
          pltpu.VMEM((n,), jnp.float32),
      ],
  )
  def deg_kernel(dst_hbm, hist_out, dst_v, hist_v):
    cid = lax.axis_index("c")
    sid = lax.axis_index("s")
    w = sid * NC + cid

    def zero_body(i, carry):
      hist_v[pl.ds(i * LANES, LANES)] = jnp.zeros((LANES,), jnp.float32)
      return carry

    lax.fori_loop(0, n // LANES, zero_body, 0)
    pltpu.sync_copy(dst_hbm.at[pl.ds(w * per_w, per_w)],
                    dst_v.at[pl.ds(0, per_w)])
    ones = jnp.ones((LANES,), jnp.float32)

    def body(i, carry):
      idx = dst_v[pl.ds(i * LANES, LANES)]
      plsc.addupdate_scatter(hist_v, [idx], ones)
      return carry

    lax.fori_loop(0, full, body, 0)
    if rem:
      idx = dst_v[pl.ds(full * LANES, LANES)]
      idx = jnp.clip(idx, 0, n - 1)
      m = lax.iota(jnp.int32, LANES) < rem
      plsc.addupdate_scatter(hist_v, [idx], ones, mask=m)
    pltpu.sync_copy(hist_v, hist_out.at[w])

  return deg_kernel


def _mm_body(x_ref, w_ref, hist_ref, ones_ref, y_ref, dinv_ref):
  deg = lax.dot_general(
      hist_ref[...], ones_ref[...],
      dimension_numbers=(((0,), (0,)), ((), ())),
      preferred_element_type=jnp.float32,
  )
  dinv = lax.rsqrt(deg + 1.0)
  y = jnp.dot(x_ref[...], w_ref[...], preferred_element_type=jnp.float32)
  y_ref[...] = y * dinv
  dinv_ref[...] = dinv


def _make_mm_call(n, d_in):
  nw = NC * NS
  nchunks = 4
  return pl.pallas_call(
      _mm_body,
      grid=(nchunks,),
      in_specs=[
          pl.BlockSpec((n, d_in), lambda c: (0, 0)),
          pl.BlockSpec((d_in, CHUNK), lambda c: (0, c)),
          pl.BlockSpec((nw, n), lambda c: (0, 0)),
          pl.BlockSpec((nw, 1), lambda c: (0, 0)),
      ],
      out_specs=[
          pl.BlockSpec((n, CHUNK), lambda c: (c, 0)),
          pl.BlockSpec((n, 1), lambda c: (0, 0)),
      ],
      out_shape=[
          jax.ShapeDtypeStruct((nchunks * n, CHUNK), jnp.float32),
          jax.ShapeDtypeStruct((n, 1), jnp.float32),
      ],
  )


def _make_scatter_kernel(n, e):
  eb = 80
  rows = e // eb
  tpr = rows // NS
  own = n // NS
  assert rows * eb == e and tpr * NS == rows
  assert (tpr - 2) % 3 == 0

  @functools.partial(
      pl.kernel,
      out_type=jax.ShapeDtypeStruct((4 * n, CHUNK), jnp.float32),
      mesh=_sc_mesh(),
      compiler_params=pltpu.CompilerParams(needs_layout_passes=False, use_tc_tiling_on_sc=False),
      scratch_types=[
          pltpu.VMEM((tpr * eb,), jnp.int32),
          pltpu.VMEM((eb,), jnp.int32),
          pltpu.VMEM((eb,), jnp.int32),
          pltpu.VMEM((eb,), jnp.int32),
          pltpu.VMEM((eb, CHUNK), jnp.float32),
          pltpu.VMEM((eb, CHUNK), jnp.float32),
          pltpu.VMEM((eb, CHUNK), jnp.float32),
          pltpu.VMEM_SHARED((n, CHUNK), jnp.float32),
          pltpu.SemaphoreType.DMA,
          pltpu.SemaphoreType.DMA,
          pltpu.SemaphoreType.DMA,
          pltpu.SemaphoreType.DMA,
          pltpu.SemaphoreType.DMA,
          pltpu.SemaphoreType.DMA,
          pltpu.SemaphoreType.DMA,
          pltpu.SemaphoreType.DMA,
          pltpu.SemaphoreType.DMA,
      ],
  )
  def scatter_kernel(y_hbm, src_hbm, dst_hbm, z_out,
                     src_v, db0, db1, db2, gb0, gb1, gb2, z_sh,
                     g0, g1, g2, d0, d1, d2, s0, s1, s2):
    cid = lax.axis_index("c")
    sid = lax.axis_index("s")
    row_lo = sid * tpr
    dbufs = (db0, db1, db2)
    gbufs = (gb0, gb1, gb2)
    gsems = (g0, g1, g2)
    dsems = (d0, d1, d2)
    ssems = (s0, s1, s2)
    pltpu.sync_copy(src_hbm.at[pl.ds(row_lo * eb, tpr * eb)], src_v)

    def dstload(j, r):
      pltpu.async_copy(dst_hbm.at[pl.ds((row_lo + j) * eb, eb)],
                       dbufs[r], dsems[r])

    def dwait(r):
      pltpu.make_async_copy(dst_hbm.at[pl.ds(0, eb)],
                            dbufs[r], dsems[r]).wait()

    def gwait(r):
      pltpu.make_async_copy(y_hbm.at[pl.ds(0, eb)], gbufs[r], gsems[r]).wait()

    def swait(r):
      pltpu.make_async_copy(y_hbm.at[pl.ds(0, eb)], gbufs[r], ssems[r]).wait()

    for p in range(2):
      c = 2 * cid + p
      base = c * n
      table = y_hbm.at[pl.ds(base, n)]

      def gather(j, r):
        pltpu.async_copy(table.at[src_v.at[pl.ds(j * eb, eb)]],
                         gbufs[r], gsems[r])

      def scatter(r):
        pltpu.async_copy(gbufs[r], z_sh.at[dbufs[r]], ssems[r], add=True)

      pltpu.sync_copy(y_hbm.at[pl.ds(base + sid * own, own)],
                      z_sh.at[pl.ds(sid * own, own)])
      plsc.subcore_barrier()

      gather(0, 0)
      dstload(0, 0)
      gather(1, 1)
      dstload(1, 1)
      gather(2, 2)
      dstload(2, 2)
      gwait(0)
      dwait(0)
      scatter(0)
      swait(0)
      gather(3, 0)
      dstload(3, 0)
      gwait(1)
      dwait(1)
      scatter(1)

      def body(k, carry):
        jb = 2 + 3 * k
        for m in range(3):
          j = jb + m
          r = (2 + m) % 3
          rn = (r + 2) % 3
          swait(rn)

          @pl.when(j + 2 < tpr)
          def _():
            gather(j + 2, rn)
            dstload(j + 2, rn)

          gwait(r)
          dwait(r)
          scatter(r)
        return carry

      lax.fori_loop(0, (tpr - 2) // 3, body, 0)
      swait((tpr - 1) % 3)
      plsc.subcore_barrier()
      pltpu.sync_copy(z_sh.at[pl.ds(sid * own, own)],
                      z_out.at[pl.ds(base + sid * own, own)])

  return scatter_kernel


def _final_body(z_ref, dinv_ref, bmu_ref, bls_ref, mu_ref, ls_ref):
  z = z_ref[...]
  dinv = dinv_ref[...]
  mu_ref[...] = jnp.concatenate([z[0], z[1]], axis=-1) * dinv + bmu_ref[...]
  ls_ref[...] = jnp.concatenate([z[2], z[3]], axis=-1) * dinv + bls_ref[...]


def _make_final_call(n, d_out):
  bm = 1000
  return pl.pallas_call(
      _final_body,
      grid=(n // bm,),
      in_specs=[
          pl.BlockSpec((4, bm, CHUNK), lambda i: (0, i, 0)),
          pl.BlockSpec((bm, 1), lambda i: (i, 0)),
          pl.BlockSpec((1, d_out), lambda i: (0, 0)),
          pl.BlockSpec((1, d_out), lambda i: (0, 0)),
      ],
      out_specs=[
          pl.BlockSpec((bm, d_out), lambda i: (i, 0)),
          pl.BlockSpec((bm, d_out), lambda i: (i, 0)),
      ],
      out_shape=[
          jax.ShapeDtypeStruct((n, d_out), jnp.float32),
          jax.ShapeDtypeStruct((n, d_out), jnp.float32),
      ],
  )


def kernel(x, edge_index, W_mu, b_mu, W_logstd, b_logstd):
  n, d_in = x.shape
  e = edge_index.shape[1]
  d_out = W_mu.shape[1]
  nw = NC * NS

  src = edge_index[0]
  dst = edge_index[1]
  w_cat = jnp.concatenate([W_mu, W_logstd], axis=1)

  hist = _make_deg_kernel(n, e)(dst)
  ones = jnp.ones((nw, 1), jnp.float32)
  y_flat, dinv = _make_mm_call(n, d_in)(x, w_cat, hist, ones)
  z_flat = _make_scatter_kernel(n, e)(y_flat, src, dst)
  z4 = z_flat.reshape(4, n, CHUNK)
  bmu2 = b_mu.reshape(1, d_out)
  bls2 = b_logstd.reshape(1, d_out)
  mu, logstd = _make_final_call(n, d_out)(z4, dinv, bmu2, bls2)
  return (mu, logstd)
